# baseline (speedup 1.0000x reference)
import functools

import jax
import jax.numpy as jnp
from jax.experimental import pallas as pl
from jax.experimental.pallas import tpu as pltpu

_N = 4194304
_COLS = 128
_ROWS = _N // _COLS
_BR = 16384
_GRID = _ROWS // _BR


def _tc_body(y_ref, s_ref, out_ref):
    i = pl.program_id(0)
    yb = y_ref[...]
    sf = s_ref[...].astype(jnp.float32)
    tot = jnp.sum(yb.reshape(_BR // 8, 8, _COLS), axis=0)
    s1 = jnp.sum((yb * sf).reshape(_BR // 8, 8, _COLS), axis=0)
    c1 = jnp.sum(sf.reshape(_BR // 8, 8, _COLS), axis=0)

    @pl.when(i == 0)
    def _init():
        out_ref[0] = tot
        out_ref[1] = s1
        out_ref[2] = c1

    @pl.when(i > 0)
    def _acc():
        out_ref[0] += tot
        out_ref[1] += s1
        out_ref[2] += c1


_tc_reduce = pl.pallas_call(
    _tc_body,
    grid=(_GRID,),
    in_specs=[
        pl.BlockSpec((_BR, _COLS), lambda i: (i, 0)),
        pl.BlockSpec((_BR, _COLS), lambda i: (i, 0)),
    ],
    out_specs=pl.BlockSpec((3, 8, _COLS), lambda i: (0, 0, 0)),
    out_shape=jax.ShapeDtypeStruct((3, 8, _COLS), jnp.float32),
    compiler_params=pltpu.CompilerParams(
        dimension_semantics=("arbitrary",),
    ),
)


def kernel(y_pred, s, y_gt):
    del y_gt
    y2 = y_pred.reshape(_ROWS, _COLS)
    s2 = s.reshape(_ROWS, _COLS)
    parts = _tc_reduce(y2, s2)
    total = jnp.sum(parts[0])
    sum1 = jnp.sum(parts[1])
    c1 = jnp.sum(parts[2])
    c0 = jnp.float32(_N) - c1
    sum0 = total - sum1
    reg_loss = jnp.abs(sum0 / c0 - sum1 / c1)
    zero = jnp.zeros((1,), dtype=jnp.float32)
    return (reg_loss, zero, zero, zero)


# speedup vs baseline: 1.3940x; 1.3940x over previous
import jax
import jax.numpy as jnp
from jax.experimental import pallas as pl
from jax.experimental.pallas import tpu as pltpu

_N = 4194304
_COLS = 128
_ROWS = _N // _COLS
_BR = 8192
_GRID = _ROWS // _BR


def _tc_body(y_ref, s_ref, out_ref, acc_ref):
    i = pl.program_id(0)
    yb = y_ref[...]
    sf = s_ref[...].astype(jnp.float32)
    tot = jnp.sum(yb.reshape(_BR // 8, 8, _COLS), axis=0)
    s1 = jnp.sum((yb * sf).reshape(_BR // 8, 8, _COLS), axis=0)
    c1 = jnp.sum(sf.reshape(_BR // 8, 8, _COLS), axis=0)

    @pl.when(i == 0)
    def _init():
        acc_ref[0] = tot
        acc_ref[1] = s1
        acc_ref[2] = c1

    @pl.when(i > 0)
    def _acc():
        acc_ref[0] += tot
        acc_ref[1] += s1
        acc_ref[2] += c1

    @pl.when(i == _GRID - 1)
    def _finalize():
        total = jnp.sum(acc_ref[0])
        sum1 = jnp.sum(acc_ref[1])
        c1t = jnp.sum(acc_ref[2])
        c0t = jnp.float32(_N) - c1t
        sum0 = total - sum1
        out_ref[0, 0] = jnp.abs(sum0 / c0t - sum1 / c1t)


_tc_reduce = pl.pallas_call(
    _tc_body,
    grid=(_GRID,),
    in_specs=[
        pl.BlockSpec((_BR, _COLS), lambda i: (i, 0)),
        pl.BlockSpec((_BR, _COLS), lambda i: (i, 0)),
    ],
    out_specs=pl.BlockSpec(memory_space=pltpu.SMEM),
    out_shape=jax.ShapeDtypeStruct((1, 1), jnp.float32),
    scratch_shapes=[pltpu.VMEM((3, 8, _COLS), jnp.float32)],
    compiler_params=pltpu.CompilerParams(
        dimension_semantics=("arbitrary",),
    ),
)


def kernel(y_pred, s, y_gt):
    del y_gt
    y2 = y_pred.reshape(_ROWS, _COLS)
    s2 = s.reshape(_ROWS, _COLS)
    reg_loss = _tc_reduce(y2, s2)[0, 0]
    zero = jnp.zeros((1,), dtype=jnp.float32)
    return (reg_loss, zero, zero, zero)
